# baseline (speedup 1.0000x reference)
import functools

import jax
import jax.experimental.layout
import jax.numpy as jnp
from jax import lax
from jax.experimental import pallas as pl
from jax.experimental.pallas import tpu as pltpu
from jax.experimental.pallas import tpu_sc as plsc

_LAMDA = 0.9
_B = 16384
_C = 1000
_NC = 2
_NS = 16
_NW = _NC * _NS
_BPW = _B // _NW
_L = 16
_CHUNKS = _BPW // _L


def _loss_body(flat_hbm, action_hbm, delta_hbm, prop_hbm, out_hbm,
               act_v, delta_v, prop_v, idx_v, sel_v, part_v, sem, sem_in):
    cid = lax.axis_index("c")
    sid = lax.axis_index("s")
    wid = sid * _NC + cid
    base = pl.multiple_of(wid * _BPW, _BPW)

    dp_copies = [
        pltpu.async_copy(delta_hbm.at[pl.ds(base, _BPW)], delta_v, sem_in),
        pltpu.async_copy(prop_hbm.at[pl.ds(base, _BPW)], prop_v, sem_in),
    ]
    pltpu.sync_copy(action_hbm.at[pl.ds(base, _BPW)], act_v)

    lane = lax.iota(jnp.int32, _L)

    def _idx_body(j, carry):
        sl = pl.ds(j * _L, _L)
        a = act_v[sl]
        r = (base + j * _L) + lane
        wa = ((lax.shift_right_logical(a, 3) * 1024
               + lax.shift_right_logical(r, 7) * 8
               + jnp.bitwise_and(a, 7)) * 128
              + jnp.bitwise_and(r, 127))
        idx_v[sl] = wa
        return carry

    half = _BPW // 2
    lax.fori_loop(0, _CHUNKS // 2, _idx_body, 0, unroll=8)
    g0 = pltpu.async_copy(flat_hbm.at[idx_v.at[pl.ds(0, half)]],
                          sel_v.at[pl.ds(0, half)], sem)
    lax.fori_loop(_CHUNKS // 2, _CHUNKS, _idx_body, 0, unroll=8)
    g1 = pltpu.async_copy(flat_hbm.at[idx_v.at[pl.ds(half, half)]],
                          sel_v.at[pl.ds(half, half)], sem)
    for c in dp_copies:
        c.wait()
    g0.wait()
    g1.wait()

    def _acc_body(j, acc):
        sl = pl.ds(j * _L, _L)
        return acc + (-delta_v[sl] - _LAMDA) * (sel_v[sl] / prop_v[sl])

    acc = lax.fori_loop(0, _CHUNKS, _acc_body,
                        jnp.zeros((_L,), jnp.float32), unroll=8)
    part_v[...] = acc * (1.0 / _B)

    pltpu.sync_copy(part_v, out_hbm.at[wid])


@functools.partial(
    pl.kernel,
    out_type=jax.ShapeDtypeStruct((_NW, _L), jnp.float32),
    mesh=plsc.VectorSubcoreMesh(
        core_axis_name="c", subcore_axis_name="s",
        num_cores=_NC, num_subcores=_NS,
    ),
    scratch_types=[
        pltpu.VMEM((_BPW,), jnp.int32),
        pltpu.VMEM((_BPW,), jnp.float32),
        pltpu.VMEM((_BPW,), jnp.float32),
        pltpu.VMEM((_BPW,), jnp.int32),
        pltpu.VMEM((_BPW,), jnp.float32),
        pltpu.VMEM((_L,), jnp.float32),
        pltpu.SemaphoreType.DMA,
        pltpu.SemaphoreType.DMA,
    ],
    compiler_params=pltpu.CompilerParams(needs_layout_passes=False),
)
def _sc_loss(flat_hbm, action_hbm, delta_hbm, prop_hbm, out_hbm,
             act_v, delta_v, prop_v, idx_v, sel_v, part_v, sem, sem_in):
    _loss_body(flat_hbm, action_hbm, delta_hbm, prop_hbm, out_hbm,
               act_v, delta_v, prop_v, idx_v, sel_v, part_v, sem, sem_in)


@jax.jit
def kernel(output, action, delta, prop):
    t = output.T.reshape(_C // 8, 8, _B // 128, 128)
    flat = t.transpose(0, 2, 1, 3).reshape(_B * _C)
    act = action.astype(jnp.int32)
    parts = _sc_loss(flat, act, delta, prop)
    return jnp.sum(parts)


# speedup vs baseline: 1.0055x; 1.0055x over previous
import functools

import jax
import jax.experimental.layout
import jax.numpy as jnp
from jax import lax
from jax.experimental import pallas as pl
from jax.experimental.pallas import tpu as pltpu
from jax.experimental.pallas import tpu_sc as plsc

_LAMDA = 0.9
_B = 16384
_C = 1000
_NC = 2
_NS = 16
_NW = _NC * _NS
_BPW = _B // _NW
_L = 16
_CHUNKS = _BPW // _L


def _loss_body(flat_hbm, action_hbm, delta_hbm, prop_hbm, out_hbm,
               act_v, delta_v, prop_v, idx_v, sel_v, part_v, sem, sem_in):
    cid = lax.axis_index("c")
    sid = lax.axis_index("s")
    wid = sid * _NC + cid
    base = pl.multiple_of(wid * _BPW, _BPW)

    dp_copies = [
        pltpu.async_copy(delta_hbm.at[pl.ds(base, _BPW)], delta_v, sem_in),
        pltpu.async_copy(prop_hbm.at[pl.ds(base, _BPW)], prop_v, sem_in),
    ]
    pltpu.sync_copy(action_hbm.at[pl.ds(base, _BPW)], act_v)

    lane = lax.iota(jnp.int32, _L)

    def _idx_body(j, carry):
        sl = pl.ds(j * _L, _L)
        a = act_v[sl]
        r = (base + j * _L) + lane
        wa = ((lax.shift_right_logical(a, 3) * 1024
               + lax.shift_right_logical(r, 7) * 8
               + jnp.bitwise_and(a, 7)) * 128
              + jnp.bitwise_and(r, 127))
        idx_v[sl] = wa
        return carry

    lax.fori_loop(0, _CHUNKS, _idx_body, 0, unroll=8)

    gather = pltpu.async_copy(flat_hbm.at[idx_v], sel_v, sem)
    for c in dp_copies:
        c.wait()
    gather.wait()

    def _acc_body(j, acc):
        sl = pl.ds(j * _L, _L)
        return acc + (-delta_v[sl] - _LAMDA) * (sel_v[sl] / prop_v[sl])

    acc = lax.fori_loop(0, _CHUNKS, _acc_body,
                        jnp.zeros((_L,), jnp.float32), unroll=8)
    part_v[...] = acc * (1.0 / _B)

    pltpu.sync_copy(part_v, out_hbm.at[wid])


@functools.partial(
    pl.kernel,
    out_type=jax.ShapeDtypeStruct((_NW, _L), jnp.float32),
    mesh=plsc.VectorSubcoreMesh(
        core_axis_name="c", subcore_axis_name="s",
        num_cores=_NC, num_subcores=_NS,
    ),
    scratch_types=[
        pltpu.VMEM((_BPW,), jnp.int32),
        pltpu.VMEM((_BPW,), jnp.float32),
        pltpu.VMEM((_BPW,), jnp.float32),
        pltpu.VMEM((_BPW,), jnp.int32),
        pltpu.VMEM((_BPW,), jnp.float32),
        pltpu.VMEM((_L,), jnp.float32),
        pltpu.SemaphoreType.DMA,
        pltpu.SemaphoreType.DMA,
    ],
    compiler_params=pltpu.CompilerParams(needs_layout_passes=False),
)
def _sc_loss(flat_hbm, action_hbm, delta_hbm, prop_hbm, out_hbm,
             act_v, delta_v, prop_v, idx_v, sel_v, part_v, sem, sem_in):
    _loss_body(flat_hbm, action_hbm, delta_hbm, prop_hbm, out_hbm,
               act_v, delta_v, prop_v, idx_v, sel_v, part_v, sem, sem_in)


@jax.jit
def kernel(output, action, delta, prop):
    t = output.T.reshape(_C // 8, 8, _B // 128, 128)
    flat = t.transpose(0, 2, 1, 3).reshape(_B * _C)
    act = action.astype(jnp.int32)
    parts = _sc_loss(flat, act, delta, prop)
    return jnp.sum(parts)


# speedup vs baseline: 1.0091x; 1.0035x over previous
import functools

import jax
import jax.numpy as jnp
from jax import lax
from jax.experimental import pallas as pl
from jax.experimental.pallas import tpu as pltpu
from jax.experimental.pallas import tpu_sc as plsc

_LAMDA = 0.9
_B = 16384
_C = 1000
_NC = 2
_NS = 16
_NW = _NC * _NS
_BPW = _B // _NW
_L = 16
_CHUNKS = _BPW // _L


def _loss_body(flat_hbm, action_hbm, delta_hbm, prop_hbm, out_hbm,
               act_v, delta_v, prop_v, idx_v, sel_v, part_v, sem, sem_in):
    cid = lax.axis_index("c")
    sid = lax.axis_index("s")
    wid = sid * _NC + cid
    base = pl.multiple_of(wid * _BPW, _BPW)

    dp_copies = [
        pltpu.async_copy(delta_hbm.at[pl.ds(base, _BPW)], delta_v, sem_in),
        pltpu.async_copy(prop_hbm.at[pl.ds(base, _BPW)], prop_v, sem_in),
    ]
    pltpu.sync_copy(action_hbm.at[pl.ds(base, _BPW)], act_v)

    lane = lax.iota(jnp.int32, _L)

    def _idx_body(j, carry):
        sl = pl.ds(j * _L, _L)
        a = act_v[sl]
        r = (base + j * _L) + lane
        wa = ((lax.shift_right_logical(a, 3) * 1024
               + lax.shift_right_logical(r, 7) * 8
               + jnp.bitwise_and(a, 7)) * 128
              + jnp.bitwise_and(r, 127))
        idx_v[sl] = wa
        return carry

    lax.fori_loop(0, _CHUNKS, _idx_body, 0, unroll=4)

    gather = pltpu.async_copy(flat_hbm.at[idx_v], sel_v, sem)
    for c in dp_copies:
        c.wait()
    gather.wait()

    def _acc_body(j, acc):
        sl = pl.ds(j * _L, _L)
        return acc + (-delta_v[sl] - _LAMDA) * (sel_v[sl] / prop_v[sl])

    acc = lax.fori_loop(0, _CHUNKS, _acc_body,
                        jnp.zeros((_L,), jnp.float32), unroll=4)
    part_v[...] = acc * (1.0 / _B)

    pltpu.sync_copy(part_v, out_hbm.at[wid])


@functools.partial(
    pl.kernel,
    out_type=jax.ShapeDtypeStruct((_NW, _L), jnp.float32),
    mesh=plsc.VectorSubcoreMesh(
        core_axis_name="c", subcore_axis_name="s",
        num_cores=_NC, num_subcores=_NS,
    ),
    scratch_types=[
        pltpu.VMEM((_BPW,), jnp.int32),
        pltpu.VMEM((_BPW,), jnp.float32),
        pltpu.VMEM((_BPW,), jnp.float32),
        pltpu.VMEM((_BPW,), jnp.int32),
        pltpu.VMEM((_BPW,), jnp.float32),
        pltpu.VMEM((_L,), jnp.float32),
        pltpu.SemaphoreType.DMA,
        pltpu.SemaphoreType.DMA,
    ],
    compiler_params=pltpu.CompilerParams(needs_layout_passes=False),
)
def _sc_loss(flat_hbm, action_hbm, delta_hbm, prop_hbm, out_hbm,
             act_v, delta_v, prop_v, idx_v, sel_v, part_v, sem, sem_in):
    _loss_body(flat_hbm, action_hbm, delta_hbm, prop_hbm, out_hbm,
               act_v, delta_v, prop_v, idx_v, sel_v, part_v, sem, sem_in)


@jax.jit
def kernel(output, action, delta, prop):
    t = output.T.reshape(_C // 8, 8, _B // 128, 128)
    flat = t.transpose(0, 2, 1, 3).reshape(_B * _C)
    act = action.astype(jnp.int32)
    parts = _sc_loss(flat, act, delta, prop)
    return jnp.sum(parts)


# speedup vs baseline: 1.0138x; 1.0047x over previous
import functools

import jax
import jax.numpy as jnp
from jax import lax
from jax.experimental import pallas as pl
from jax.experimental.pallas import tpu as pltpu
from jax.experimental.pallas import tpu_sc as plsc

_LAMDA = 0.9
_B = 16384
_C = 1000
_NC = 2
_NS = 16
_NW = _NC * _NS
_BPW = _B // _NW
_L = 16
_CHUNKS = _BPW // _L


def _loss_body(flat_hbm, action_hbm, delta_hbm, prop_hbm, out_hbm,
               act_v, delta_v, prop_v, idx_v, sel_v, part_v, sem, sem_in):
    cid = lax.axis_index("c")
    sid = lax.axis_index("s")
    wid = sid * _NC + cid
    base = pl.multiple_of(wid * _BPW, _BPW)

    dp_copies = [
        pltpu.async_copy(delta_hbm.at[pl.ds(base, _BPW)], delta_v, sem_in),
        pltpu.async_copy(prop_hbm.at[pl.ds(base, _BPW)], prop_v, sem_in),
    ]
    pltpu.sync_copy(action_hbm.at[pl.ds(base, _BPW)], act_v)

    lane = lax.iota(jnp.int32, _L)

    def _idx_body(j, carry):
        sl = pl.ds(j * _L, _L)
        a = act_v[sl]
        r = (base + j * _L) + lane
        wa = ((lax.shift_right_logical(a, 3) * 1024
               + lax.shift_right_logical(r, 7) * 8
               + jnp.bitwise_and(a, 7)) * 128
              + jnp.bitwise_and(r, 127))
        idx_v[sl] = wa
        return carry

    lax.fori_loop(0, _CHUNKS, _idx_body, 0, unroll=1)

    gather = pltpu.async_copy(flat_hbm.at[idx_v], sel_v, sem)
    for c in dp_copies:
        c.wait()
    gather.wait()

    def _acc_body(j, acc):
        sl = pl.ds(j * _L, _L)
        return acc + (-delta_v[sl] - _LAMDA) * (sel_v[sl] / prop_v[sl])

    acc = lax.fori_loop(0, _CHUNKS, _acc_body,
                        jnp.zeros((_L,), jnp.float32), unroll=1)
    part_v[...] = acc * (1.0 / _B)

    pltpu.sync_copy(part_v, out_hbm.at[wid])


@functools.partial(
    pl.kernel,
    out_type=jax.ShapeDtypeStruct((_NW, _L), jnp.float32),
    mesh=plsc.VectorSubcoreMesh(
        core_axis_name="c", subcore_axis_name="s",
        num_cores=_NC, num_subcores=_NS,
    ),
    scratch_types=[
        pltpu.VMEM((_BPW,), jnp.int32),
        pltpu.VMEM((_BPW,), jnp.float32),
        pltpu.VMEM((_BPW,), jnp.float32),
        pltpu.VMEM((_BPW,), jnp.int32),
        pltpu.VMEM((_BPW,), jnp.float32),
        pltpu.VMEM((_L,), jnp.float32),
        pltpu.SemaphoreType.DMA,
        pltpu.SemaphoreType.DMA,
    ],
    compiler_params=pltpu.CompilerParams(needs_layout_passes=False),
)
def _sc_loss(flat_hbm, action_hbm, delta_hbm, prop_hbm, out_hbm,
             act_v, delta_v, prop_v, idx_v, sel_v, part_v, sem, sem_in):
    _loss_body(flat_hbm, action_hbm, delta_hbm, prop_hbm, out_hbm,
               act_v, delta_v, prop_v, idx_v, sel_v, part_v, sem, sem_in)


@jax.jit
def kernel(output, action, delta, prop):
    t = output.T.reshape(_C // 8, 8, _B // 128, 128)
    flat = t.transpose(0, 2, 1, 3).reshape(_B * _C)
    act = action.astype(jnp.int32)
    parts = _sc_loss(flat, act, delta, prop)
    return jnp.sum(parts)


# speedup vs baseline: 1.0165x; 1.0027x over previous
import functools

import jax
import jax.numpy as jnp
from jax import lax
from jax.experimental import pallas as pl
from jax.experimental.pallas import tpu as pltpu
from jax.experimental.pallas import tpu_sc as plsc

_LAMDA = 0.9
_B = 16384
_C = 1000
_NC = 2
_NS = 16
_NW = _NC * _NS
_BPW = _B // _NW
_L = 16
_CHUNKS = _BPW // _L


def _loss_body(flat_hbm, action_hbm, delta_hbm, prop_hbm, out_hbm,
               act_v, delta_v, prop_v, idx_v, sel_v, part_v, sem, sem_in):
    cid = lax.axis_index("c")
    sid = lax.axis_index("s")
    wid = sid * _NC + cid
    base = pl.multiple_of(wid * _BPW, _BPW)

    dp_copies = [
        pltpu.async_copy(delta_hbm.at[pl.ds(base, _BPW)], delta_v, sem_in),
        pltpu.async_copy(prop_hbm.at[pl.ds(base, _BPW)], prop_v, sem_in),
    ]
    pltpu.sync_copy(action_hbm.at[pl.ds(base, _BPW)], act_v)

    lane = lax.iota(jnp.int32, _L)

    def _idx_body(j, carry):
        sl = pl.ds(j * _L, _L)
        a = act_v[sl]
        r = (base + j * _L) + lane
        wa = ((lax.shift_right_logical(a, 3) * 1024
               + lax.shift_right_logical(r, 7) * 8
               + jnp.bitwise_and(a, 7)) * 128
              + jnp.bitwise_and(r, 127))
        idx_v[sl] = wa
        return carry

    lax.fori_loop(0, _CHUNKS, _idx_body, 0, unroll=1)

    gather = pltpu.async_copy(flat_hbm.at[idx_v], sel_v, sem)
    for c in dp_copies:
        c.wait()
    gather.wait()

    def _acc_body(j, acc):
        sl = pl.ds(j * _L, _L)
        return acc + (-delta_v[sl] - _LAMDA) * (sel_v[sl] / prop_v[sl])

    acc = lax.fori_loop(0, _CHUNKS, _acc_body,
                        jnp.zeros((_L,), jnp.float32), unroll=1)
    part_v[...] = acc * (1.0 / _B)

    pltpu.sync_copy(part_v, out_hbm.at[wid])


@functools.partial(
    pl.kernel,
    out_type=jax.ShapeDtypeStruct((_NW, _L), jnp.float32),
    mesh=plsc.VectorSubcoreMesh(
        core_axis_name="c", subcore_axis_name="s",
        num_cores=_NC, num_subcores=_NS,
    ),
    scratch_types=[
        pltpu.VMEM((_BPW,), jnp.int32),
        pltpu.VMEM((_BPW,), jnp.float32),
        pltpu.VMEM((_BPW,), jnp.float32),
        pltpu.VMEM((_BPW,), jnp.int32),
        pltpu.VMEM((_BPW,), jnp.float32),
        pltpu.VMEM((_L,), jnp.float32),
        pltpu.SemaphoreType.DMA,
        pltpu.SemaphoreType.DMA,
    ],
    compiler_params=pltpu.CompilerParams(needs_layout_passes=False, skip_device_barrier=True),
)
def _sc_loss(flat_hbm, action_hbm, delta_hbm, prop_hbm, out_hbm,
             act_v, delta_v, prop_v, idx_v, sel_v, part_v, sem, sem_in):
    _loss_body(flat_hbm, action_hbm, delta_hbm, prop_hbm, out_hbm,
               act_v, delta_v, prop_v, idx_v, sel_v, part_v, sem, sem_in)


@jax.jit
def kernel(output, action, delta, prop):
    t = output.T.reshape(_C // 8, 8, _B // 128, 128)
    flat = t.transpose(0, 2, 1, 3).reshape(_B * _C)
    act = action.astype(jnp.int32)
    parts = _sc_loss(flat, act, delta, prop)
    return jnp.sum(parts)


# speedup vs baseline: 1.0169x; 1.0004x over previous
import functools

import jax
import jax.numpy as jnp
from jax import lax
from jax.experimental import pallas as pl
from jax.experimental.pallas import tpu as pltpu
from jax.experimental.pallas import tpu_sc as plsc

_LAMDA = 0.9
_B = 16384
_C = 1000
_NC = 2
_NS = 16
_NW = _NC * _NS
_BPW = _B // _NW
_L = 16
_CHUNKS = _BPW // _L


def _loss_body(flat_hbm, action_hbm, delta_hbm, prop_hbm, out_hbm,
               act_v, delta_v, prop_v, idx_v, sel_v, part_v, sem, sem_in):
    cid = lax.axis_index("c")
    sid = lax.axis_index("s")
    wid = sid * _NC + cid
    base = pl.multiple_of(wid * _BPW, _BPW)

    dp_copies = [
        pltpu.async_copy(delta_hbm.at[pl.ds(base, _BPW)], delta_v, sem_in),
        pltpu.async_copy(prop_hbm.at[pl.ds(base, _BPW)], prop_v, sem_in),
    ]
    pltpu.sync_copy(action_hbm.at[pl.ds(base, _BPW)], act_v)

    lane = lax.iota(jnp.int32, _L)

    def _idx_body(j, carry):
        sl = pl.ds(j * _L, _L)
        a = act_v[sl]
        r = (base + j * _L) + lane
        wa = ((lax.shift_right_logical(a, 3) * 1024
               + lax.shift_right_logical(r, 7) * 8
               + jnp.bitwise_and(a, 7)) * 128
              + jnp.bitwise_and(r, 127))
        idx_v[sl] = wa
        return carry

    lax.fori_loop(0, _CHUNKS, _idx_body, 0, unroll=1)

    gather = pltpu.async_copy(flat_hbm.at[idx_v], sel_v, sem)
    for c in dp_copies:
        c.wait()
    gather.wait()

    def _acc_body(j, acc):
        sl = pl.ds(j * _L, _L)
        return acc + (-delta_v[sl] - _LAMDA) * (sel_v[sl] / prop_v[sl])

    acc = lax.fori_loop(0, _CHUNKS, _acc_body,
                        jnp.zeros((_L,), jnp.float32), unroll=1)
    part_v[...] = acc * (1.0 / _B)

    pltpu.sync_copy(part_v, out_hbm.at[wid])


@functools.partial(
    pl.kernel,
    out_type=jax.ShapeDtypeStruct((_NW, _L), jnp.float32),
    mesh=plsc.VectorSubcoreMesh(
        core_axis_name="c", subcore_axis_name="s",
        num_cores=_NC, num_subcores=_NS,
    ),
    scratch_types=[
        pltpu.VMEM((_BPW,), jnp.int32),
        pltpu.VMEM((_BPW,), jnp.float32),
        pltpu.VMEM((_BPW,), jnp.float32),
        pltpu.VMEM((_BPW,), jnp.int32),
        pltpu.VMEM((_BPW,), jnp.float32),
        pltpu.VMEM((_L,), jnp.float32),
        pltpu.SemaphoreType.DMA,
        pltpu.SemaphoreType.DMA,
    ],
    compiler_params=pltpu.CompilerParams(needs_layout_passes=False),
)
def _sc_loss(flat_hbm, action_hbm, delta_hbm, prop_hbm, out_hbm,
             act_v, delta_v, prop_v, idx_v, sel_v, part_v, sem, sem_in):
    _loss_body(flat_hbm, action_hbm, delta_hbm, prop_hbm, out_hbm,
               act_v, delta_v, prop_v, idx_v, sel_v, part_v, sem, sem_in)


@jax.jit
def kernel(output, action, delta, prop):
    t = output.T.reshape(_C // 8, 8, _B // 128, 128)
    flat = t.transpose(0, 2, 1, 3).reshape(_B * _C)
    act = action.astype(jnp.int32)
    parts = _sc_loss(flat, act, delta, prop)
    return jnp.sum(parts)
